# baseline (speedup 1.0000x reference)
import functools

import jax
import jax.numpy as jnp
from jax import lax
from jax.experimental import pallas as pl
from jax.experimental.pallas import tpu as pltpu
from jax.experimental.pallas import tpu_sc as plsc

_NC = 2
_NS = 16
_NW = _NC * _NS
_CH = 128

_S = 50
_DT = 8
_BT = 128
_BTW = _BT // _NW
_NCHUNK = _S * _BTW


def _i16(v):
    return jax.lax.iota(jnp.int32, 16) + v


@functools.lru_cache(maxsize=None)
def _emb_call():
    mesh = plsc.VectorSubcoreMesh(core_axis_name="c", subcore_axis_name="s")

    @functools.partial(
        pl.kernel,
        mesh=mesh,
        out_type=jax.ShapeDtypeStruct((_S, _DT, _BT, 8, 128), jnp.float32),
        scratch_types=(
            [pltpu.VMEM((_S, 512), jnp.int32)]
            + [pltpu.VMEM((2, _CH), jnp.int32)]
            + [pltpu.VMEM((2, _CH), jnp.int32)]
            + [pltpu.VMEM((_CH, 128), jnp.float32) for _ in range(2)]
            + [pltpu.VMEM((_DT, 8, 128), jnp.float32) for _ in range(2)]
            + [pltpu.SemaphoreType.DMA for _ in range(4)]
        ),
        compiler_params=pltpu.CompilerParams(
            use_tc_tiling_on_sc=False, needs_layout_passes=False
        ),
    )
    def run(table, ids, out, ids_all, gidx, hv, rows0, rows1, lb0, lb1,
            g0, g1, o0, o1):
        rows = (rows0, rows1)
        lbs = (lb0, lb1)
        gsem = (g0, g1)
        osem = (o0, o1)
        wid = lax.axis_index("s") * _NC + lax.axis_index("c")
        b0 = wid * 512

        pltpu.sync_copy(ids.at[:, pl.ds(b0, 512)], ids_all)

        def prep_and_gather(c, p):
            s = c // _BTW
            j = c - s * _BTW
            base = j * _CH
            for i in range(8):
                v16 = ids_all[s, pl.ds(base + i * 16, 16)]
                gidx[p, pl.ds(i * 16, 16)] = v16 >> 1
                hv[p, pl.ds(i * 16, 16)] = (v16 & 1) << 6
            pltpu.async_copy(table.at[gidx.at[p]], rows[p], gsem[p])

        def assemble(p):
            half = tuple(hv[p, pl.ds(i * 16, 16)] for i in range(8))
            bc = tuple(_i16(i * 16) for i in range(8))

            def body(dt, carry):
                for dr in range(8):
                    dv = jnp.full((16,), dt * 8 + dr, dtype=jnp.int32)
                    for i in range(8):
                        val = plsc.load_gather(rows[p], [bc[i], carry[i] + dv])
                        lbs[p][dt, dr, pl.ds(i * 16, 16)] = val
                return carry

            lax.fori_loop(0, _DT, body, half)

        def out_slice(c):
            s = c // _BTW
            j = c - s * _BTW
            return out.at[s, :, wid * _BTW + j]

        for p in range(2):
            prep_and_gather(p, p)

        def step(g, carry):
            for p in range(2):
                c = 2 * g + p
                pltpu.make_async_copy(table.at[gidx.at[p]], rows[p],
                                      gsem[p]).wait()

                @pl.when(g >= 1)
                def _():
                    pltpu.make_async_copy(lbs[p], out_slice(c - 2),
                                          osem[p]).wait()

                assemble(p)
                pltpu.async_copy(lbs[p], out_slice(c), osem[p])

                @pl.when(c + 2 < _NCHUNK)
                def _():
                    prep_and_gather(c + 2, p)

            return carry

        lax.fori_loop(0, _NCHUNK // 2, step, 0)

        for p in range(2):
            pltpu.make_async_copy(lbs[p], out_slice(_NCHUNK - 2 + p),
                                  osem[p]).wait()

    return run


def kernel(token_ids, W):
    W2 = W.reshape(500000, 128)
    ids_t = token_ids.T.astype(jnp.int32)
    L = _emb_call()(W2, ids_t)
    return jnp.transpose(L, (2, 4, 0, 1, 3)).reshape(16384, 50, 64)
